# baseline (speedup 1.0000x reference)
import functools

import jax
import jax.numpy as jnp
from jax import lax
from jax.experimental import pallas as pl
from jax.experimental.pallas import tpu as pltpu
from jax.experimental.pallas import tpu_sc as plsc

BATCH = 4096
HIST = 200
DIM = 64
TOTAL = BATCH * HIST
NW = 32
PER_W = TOTAL // NW
CHUNK = 128
NCHUNK = PER_W // CHUNK
NBUF = 4
NGROUP = NCHUNK // NBUF

_mesh = plsc.VectorSubcoreMesh(core_axis_name="c", subcore_axis_name="s")


def _body(table_hbm, idx_hbm, out_hbm, idx_v,
          b0, b1, b2, b3, g0, g1, g2, g3, s0, s1, s2, s3):
    bufs = (b0, b1, b2, b3)
    gsem = (g0, g1, g2, g3)
    ssem = (s0, s1, s2, s3)
    wid = lax.axis_index("s") * 2 + lax.axis_index("c")

    pltpu.sync_copy(idx_hbm.at[wid], idx_v)

    def gather_start(j, b):
        pltpu.async_copy(table_hbm.at[idx_v.at[j]], bufs[b], gsem[b])

    def gather_wait(j, b):
        pltpu.make_async_copy(table_hbm.at[idx_v.at[j]], bufs[b], gsem[b]).wait()

    def store_start(j, b):
        pltpu.async_copy(bufs[b], out_hbm.at[wid, j], ssem[b])

    def store_wait(j, b):
        pltpu.make_async_copy(bufs[b], out_hbm.at[wid, j], ssem[b]).wait()

    for b in range(NBUF):
        gather_start(b, b)

    def group(g, carry):
        for b in range(NBUF):
            j = g * NBUF + b
            gather_wait(j, b)
            store_start(j, b)
            store_wait(j, b)
            gather_start(j + NBUF, b)
        return carry

    lax.fori_loop(0, NGROUP - 1, group, 0)

    for b in range(NBUF):
        j = (NGROUP - 1) * NBUF + b
        gather_wait(j, b)
        store_start(j, b)
    for b in range(NBUF):
        j = (NGROUP - 1) * NBUF + b
        store_wait(j, b)


_call = functools.partial(
    pl.kernel,
    mesh=_mesh,
    compiler_params=pltpu.CompilerParams(use_tc_tiling_on_sc=False),
    out_type=jax.ShapeDtypeStruct((NW, NCHUNK, CHUNK, DIM), jnp.float32),
    scratch_types=(
        [pltpu.VMEM((NCHUNK, CHUNK), jnp.int32)]
        + [pltpu.VMEM((CHUNK, DIM), jnp.float32)] * NBUF
        + [pltpu.SemaphoreType.DMA] * (2 * NBUF)
    ),
)(_body)


def kernel(token_ids, embedding):
    idx = token_ids.reshape(NW, NCHUNK, CHUNK).astype(jnp.int32)
    out = _call(embedding, idx)
    return out.reshape(BATCH, HIST, DIM)
